# baseline (speedup 1.0000x reference)
import functools

import jax
import jax.numpy as jnp
from jax import lax
from jax.experimental import pallas as pl
from jax.experimental.pallas import tpu as pltpu
from jax.experimental.pallas import tpu_sc as plsc

T = 16384
D = 2048
E = 16
K = 2
TB = 512
NBUF = 8
NBLK = T // TB

NW = 32
TPW = T // NW


def _copy(x_hbm, buf_ref, sem, blk, slot):
    return pltpu.make_async_copy(
        x_hbm.at[pl.ds(blk * TB, TB), :], buf_ref.at[slot], sem.at[slot])


def _gemm_body(x_hbm, w_ref, b_ref, logits_ref, buf_ref, sem):
    i = pl.program_id(0)

    @pl.when(i == 0)
    def _prime():
        for b in range(NBUF - 1):
            _copy(x_hbm, buf_ref, sem, b, b).start()

    nxt = i + NBUF - 1

    @pl.when(nxt < NBLK)
    def _refill():
        _copy(x_hbm, buf_ref, sem, nxt, nxt % NBUF).start()

    slot = jax.lax.rem(i, NBUF)
    _copy(x_hbm, buf_ref, sem, i, slot).wait()

    logits_ref[...] = jnp.dot(buf_ref[slot], w_ref[...],
                              preferred_element_type=jnp.float32) + b_ref[...]


def _sc_routing_body(logits_hbm, tkw_hbm, tki_hbm, cnt_hbm, psum_hbm,
                     lvm, owf, owi, pvm, sem):
    wid = lax.axis_index("s") * 2 + lax.axis_index("c")
    base = wid * TPW
    pltpu.async_copy(logits_hbm.at[pl.ds(base, TPW), :], lvm, sem).wait()

    lane_i = lax.iota(jnp.int32, 16)
    lane = lane_i.astype(jnp.float32)
    zeros = jnp.zeros((16,), jnp.float32)

    def group(g, carry):
        cnt_acc, ps_acc = carry
        w1v = zeros
        w2v = zeros
        i1v = zeros
        i2v = zeros
        for j in range(16):
            v = lvm[g * 16 + j, :]
            m1 = jnp.max(v)
            i1 = jnp.min(jnp.where(v == m1, lane, 16.0))
            hit1 = lane == i1
            v2 = jnp.where(hit1, -jnp.inf, v)
            m2 = jnp.max(v2)
            i2 = jnp.min(jnp.where(v2 == m2, lane, 16.0))
            hit2 = lane == i2
            e = jnp.exp(v - m1)
            s = jnp.sum(e)
            pv = e / s
            ps_acc = ps_acc + pv
            cnt_acc = (cnt_acc + jnp.where(hit1, 1.0, 0.0)
                       + jnp.where(hit2, 1.0, 0.0))
            mine = lane_i == j
            w1v = jnp.where(mine, jnp.sum(jnp.where(hit1, pv, zeros)), w1v)
            w2v = jnp.where(mine, jnp.sum(jnp.where(hit2, pv, zeros)), w2v)
            i1v = jnp.where(mine, i1, i1v)
            i2v = jnp.where(mine, i2, i2v)
        owf[0, pl.ds(g * 16, 16)] = w1v
        owf[1, pl.ds(g * 16, 16)] = w2v
        owi[0, pl.ds(g * 16, 16)] = i1v.astype(jnp.int32)
        owi[1, pl.ds(g * 16, 16)] = i2v.astype(jnp.int32)
        return cnt_acc, ps_acc

    cnt, ps = lax.fori_loop(0, TPW // 16, group, (zeros, zeros))

    pvm[0, :] = cnt
    pvm[1, :] = ps
    pltpu.sync_copy(owf, tkw_hbm.at[:, pl.ds(base, TPW)])
    pltpu.sync_copy(owi, tki_hbm.at[:, pl.ds(base, TPW)])
    pltpu.sync_copy(pvm.at[0], cnt_hbm.at[wid])
    pltpu.sync_copy(pvm.at[1], psum_hbm.at[wid])


@jax.jit
def _router(x_flat, W, b):
    logits = pl.pallas_call(
        _gemm_body,
        grid=(NBLK,),
        in_specs=[
            pl.BlockSpec(memory_space=pl.ANY),
            pl.BlockSpec((D, E), lambda i: (0, 0)),
            pl.BlockSpec((1, E), lambda i: (0, 0)),
        ],
        out_specs=pl.BlockSpec((TB, E), lambda i: (i, 0)),
        out_shape=jax.ShapeDtypeStruct((T, E), jnp.float32),
        scratch_shapes=[
            pltpu.VMEM((NBUF, TB, D), jnp.float32),
            pltpu.SemaphoreType.DMA((NBUF,)),
        ],
    )(x_flat, W, b.reshape(1, E))

    mesh = plsc.VectorSubcoreMesh(core_axis_name="c", subcore_axis_name="s")
    sc = functools.partial(
        pl.kernel,
        out_type=[
            jax.ShapeDtypeStruct((K, T), jnp.float32),
            jax.ShapeDtypeStruct((K, T), jnp.int32),
            jax.ShapeDtypeStruct((NW, E), jnp.float32),
            jax.ShapeDtypeStruct((NW, E), jnp.float32),
        ],
        mesh=mesh,
        compiler_params=pltpu.CompilerParams(needs_layout_passes=False),
        scratch_types=[
            pltpu.VMEM((TPW, E), jnp.float32),
            pltpu.VMEM((K, TPW), jnp.float32),
            pltpu.VMEM((K, TPW), jnp.int32),
            pltpu.VMEM((2, E), jnp.float32),
            pltpu.SemaphoreType.DMA,
        ],
    )(_sc_routing_body)
    tkw_t, tki_t, cnt_p, psum_p = sc(logits)

    aux = (E / (T * T)) * jnp.sum(jnp.sum(cnt_p, axis=0)
                                  * jnp.sum(psum_p, axis=0))
    return tkw_t.T, tki_t.T.astype(jnp.int64), aux


def kernel(x_flat, W, b):
    return _router(x_flat, W, b)


# speedup vs baseline: 1.1641x; 1.1641x over previous
import jax
import jax.numpy as jnp
from jax.experimental import pallas as pl
from jax.experimental.pallas import tpu as pltpu

T = 16384
D = 2048
E = 16
K = 2
TB = 256
NBUF = 16
NBLK = T // TB


def _copy(x_hbm, buf_ref, sem, blk, slot):
    return pltpu.make_async_copy(
        x_hbm.at[pl.ds(blk * TB, TB), :], buf_ref.at[slot], sem.at[slot])


def _router_body(x_hbm, w_ref, b_ref, tkw_ref, tki_ref, cnt_ref, psum_ref,
                 aux_ref, buf_ref, sem):
    i = pl.program_id(0)
    n = pl.num_programs(0)

    @pl.when(i == 0)
    def _prime():
        for b in range(NBUF - 1):
            _copy(x_hbm, buf_ref, sem, b, b).start()

    nxt = i + NBUF - 1

    @pl.when(nxt < NBLK)
    def _refill():
        _copy(x_hbm, buf_ref, sem, nxt, nxt % NBUF).start()

    slot = jax.lax.rem(i, NBUF)
    _copy(x_hbm, buf_ref, sem, i, slot).wait()

    logits = jnp.dot(buf_ref[slot], w_ref[...],
                     preferred_element_type=jnp.float32) + b_ref[...]

    lane = jax.lax.broadcasted_iota(
        jnp.int32, (TB, E), 1).astype(jnp.float32)

    m1 = jnp.max(logits, axis=-1, keepdims=True)
    i1 = jnp.min(jnp.where(logits == m1, lane, E), axis=-1, keepdims=True)
    hit1 = lane == i1
    l2 = jnp.where(hit1, -jnp.inf, logits)
    m2 = jnp.max(l2, axis=-1, keepdims=True)
    i2 = jnp.min(jnp.where(l2 == m2, lane, E), axis=-1, keepdims=True)
    hit2 = lane == i2

    e = jnp.exp(logits - m1)
    s = jnp.sum(e, axis=-1, keepdims=True)
    r = 1.0 / s
    p = e * r

    tkw_ref[...] = jnp.concatenate([r, jnp.exp(m2 - m1) * r], axis=-1)
    tki_ref[...] = jnp.concatenate([i1, i2], axis=-1).astype(jnp.int32)

    cnt = jnp.sum((hit1 | hit2).astype(jnp.float32), axis=0, keepdims=True)
    psum = jnp.sum(p, axis=0, keepdims=True)

    @pl.when(i == 0)
    def _init():
        cnt_ref[...] = cnt
        psum_ref[...] = psum

    @pl.when(i > 0)
    def _acc():
        cnt_ref[...] += cnt
        psum_ref[...] += psum

    @pl.when(i == n - 1)
    def _fin():
        aux_ref[...] = (E / (T * T)) * jnp.sum(
            cnt_ref[...] * psum_ref[...], keepdims=True)


@jax.jit
def _router(x_flat, W, b):
    tkw, tki, _, _, aux = pl.pallas_call(
        _router_body,
        grid=(NBLK,),
        in_specs=[
            pl.BlockSpec(memory_space=pl.ANY),
            pl.BlockSpec((D, E), lambda i: (0, 0)),
            pl.BlockSpec((1, E), lambda i: (0, 0)),
        ],
        out_specs=[
            pl.BlockSpec((TB, K), lambda i: (i, 0)),
            pl.BlockSpec((TB, K), lambda i: (i, 0)),
            pl.BlockSpec((1, E), lambda i: (0, 0)),
            pl.BlockSpec((1, E), lambda i: (0, 0)),
            pl.BlockSpec((1, 1), lambda i: (0, 0)),
        ],
        out_shape=[
            jax.ShapeDtypeStruct((T, K), jnp.float32),
            jax.ShapeDtypeStruct((T, K), jnp.int32),
            jax.ShapeDtypeStruct((1, E), jnp.float32),
            jax.ShapeDtypeStruct((1, E), jnp.float32),
            jax.ShapeDtypeStruct((1, 1), jnp.float32),
        ],
        scratch_shapes=[
            pltpu.VMEM((NBUF, TB, D), jnp.float32),
            pltpu.SemaphoreType.DMA((NBUF,)),
        ],
    )(x_flat, W, b.reshape(1, E))
    return tkw, tki.astype(jnp.int64), aux[0, 0]


def kernel(x_flat, W, b):
    return _router(x_flat, W, b)


# speedup vs baseline: 1.3535x; 1.1627x over previous
import jax
import jax.numpy as jnp
from jax.experimental import pallas as pl
from jax.experimental.pallas import tpu as pltpu

T = 16384
D = 2048
E = 16
K = 2
TB = 512
NBUF = 12
NBLK = T // TB


def _copy(x_hbm, buf_ref, sem, blk, slot):
    return pltpu.make_async_copy(
        x_hbm.at[pl.ds(blk * TB, TB), :], buf_ref.at[slot], sem.at[slot])


def _router_body(x_hbm, w_ref, b_ref, tkw_ref, tki_ref, cnt_ref, psum_ref,
                 aux_ref, buf_ref, sem):
    i = pl.program_id(0)
    n = pl.num_programs(0)

    @pl.when(i == 0)
    def _prime():
        for b in range(NBUF - 1):
            _copy(x_hbm, buf_ref, sem, b, b).start()

    nxt = i + NBUF - 1

    @pl.when(nxt < NBLK)
    def _refill():
        _copy(x_hbm, buf_ref, sem, nxt, nxt % NBUF).start()

    slot = jax.lax.rem(i, NBUF)
    _copy(x_hbm, buf_ref, sem, i, slot).wait()

    logits = jnp.dot(buf_ref[slot], w_ref[...],
                     preferred_element_type=jnp.float32) + b_ref[...]

    lane = jax.lax.broadcasted_iota(
        jnp.int32, (TB, E), 1).astype(jnp.float32)

    m1 = jnp.max(logits, axis=-1, keepdims=True)
    i1 = jnp.min(jnp.where(logits == m1, lane, E), axis=-1, keepdims=True)
    hit1 = lane == i1
    l2 = jnp.where(hit1, -jnp.inf, logits)
    m2 = jnp.max(l2, axis=-1, keepdims=True)
    i2 = jnp.min(jnp.where(l2 == m2, lane, E), axis=-1, keepdims=True)
    hit2 = lane == i2

    e = jnp.exp(logits - m1)
    s = jnp.sum(e, axis=-1, keepdims=True)
    r = 1.0 / s
    p = e * r

    tkw_ref[...] = jnp.concatenate([r, jnp.exp(m2 - m1) * r], axis=-1)
    tki_ref[...] = jnp.concatenate([i1, i2], axis=-1).astype(jnp.int32)

    cnt = jnp.sum((hit1 | hit2).astype(jnp.float32), axis=0, keepdims=True)
    psum = jnp.sum(p, axis=0, keepdims=True)

    @pl.when(i == 0)
    def _init():
        cnt_ref[...] = cnt
        psum_ref[...] = psum

    @pl.when(i > 0)
    def _acc():
        cnt_ref[...] += cnt
        psum_ref[...] += psum

    @pl.when(i == n - 1)
    def _fin():
        aux_ref[...] = (E / (T * T)) * jnp.sum(
            cnt_ref[...] * psum_ref[...], keepdims=True)


@jax.jit
def _router(x_flat, W, b):
    tkw, tki, _, _, aux = pl.pallas_call(
        _router_body,
        grid=(NBLK,),
        in_specs=[
            pl.BlockSpec(memory_space=pl.ANY),
            pl.BlockSpec((D, E), lambda i: (0, 0)),
            pl.BlockSpec((1, E), lambda i: (0, 0)),
        ],
        out_specs=[
            pl.BlockSpec((TB, K), lambda i: (i, 0)),
            pl.BlockSpec((TB, K), lambda i: (i, 0)),
            pl.BlockSpec((1, E), lambda i: (0, 0)),
            pl.BlockSpec((1, E), lambda i: (0, 0)),
            pl.BlockSpec((1, 1), lambda i: (0, 0)),
        ],
        out_shape=[
            jax.ShapeDtypeStruct((T, K), jnp.float32),
            jax.ShapeDtypeStruct((T, K), jnp.int32),
            jax.ShapeDtypeStruct((1, E), jnp.float32),
            jax.ShapeDtypeStruct((1, E), jnp.float32),
            jax.ShapeDtypeStruct((1, 1), jnp.float32),
        ],
        scratch_shapes=[
            pltpu.VMEM((NBUF, TB, D), jnp.float32),
            pltpu.SemaphoreType.DMA((NBUF,)),
        ],
    )(x_flat, W, b.reshape(1, E))
    return tkw, tki.astype(jnp.int64), aux[0, 0]


def kernel(x_flat, W, b):
    return _router(x_flat, W, b)


# speedup vs baseline: 1.3582x; 1.0035x over previous
import jax
import jax.numpy as jnp
from jax.experimental import pallas as pl
from jax.experimental.pallas import tpu as pltpu

T = 16384
D = 2048
E = 16
K = 2
TB = 1024
NBUF = 6
NBLK = T // TB


def _copy(x_hbm, buf_ref, sem, blk, slot):
    return pltpu.make_async_copy(
        x_hbm.at[pl.ds(blk * TB, TB), :], buf_ref.at[slot], sem.at[slot])


def _router_body(x_hbm, w_ref, b_ref, tkw_ref, tki_ref, cnt_ref, psum_ref,
                 aux_ref, buf_ref, sem):
    i = pl.program_id(0)
    n = pl.num_programs(0)

    @pl.when(i == 0)
    def _prime():
        for b in range(NBUF - 1):
            _copy(x_hbm, buf_ref, sem, b, b).start()

    nxt = i + NBUF - 1

    @pl.when(nxt < NBLK)
    def _refill():
        _copy(x_hbm, buf_ref, sem, nxt, nxt % NBUF).start()

    slot = jax.lax.rem(i, NBUF)
    _copy(x_hbm, buf_ref, sem, i, slot).wait()

    logits = jnp.dot(buf_ref[slot], w_ref[...],
                     preferred_element_type=jnp.float32) + b_ref[...]

    lane = jax.lax.broadcasted_iota(
        jnp.int32, (TB, E), 1).astype(jnp.float32)

    m1 = jnp.max(logits, axis=-1, keepdims=True)
    i1 = jnp.min(jnp.where(logits == m1, lane, E), axis=-1, keepdims=True)
    hit1 = lane == i1
    l2 = jnp.where(hit1, -jnp.inf, logits)
    m2 = jnp.max(l2, axis=-1, keepdims=True)
    i2 = jnp.min(jnp.where(l2 == m2, lane, E), axis=-1, keepdims=True)
    hit2 = lane == i2

    e = jnp.exp(logits - m1)
    s = jnp.sum(e, axis=-1, keepdims=True)
    r = 1.0 / s
    p = e * r

    tkw_ref[...] = jnp.concatenate([r, jnp.exp(m2 - m1) * r], axis=-1)
    tki_ref[...] = jnp.concatenate([i1, i2], axis=-1).astype(jnp.int32)

    cnt = jnp.sum((hit1 | hit2).astype(jnp.float32), axis=0, keepdims=True)
    psum = jnp.sum(p, axis=0, keepdims=True)

    @pl.when(i == 0)
    def _init():
        cnt_ref[...] = cnt
        psum_ref[...] = psum

    @pl.when(i > 0)
    def _acc():
        cnt_ref[...] += cnt
        psum_ref[...] += psum

    @pl.when(i == n - 1)
    def _fin():
        aux_ref[...] = (E / (T * T)) * jnp.sum(
            cnt_ref[...] * psum_ref[...], keepdims=True)


@jax.jit
def _router(x_flat, W, b):
    tkw, tki, _, _, aux = pl.pallas_call(
        _router_body,
        grid=(NBLK,),
        in_specs=[
            pl.BlockSpec(memory_space=pl.ANY),
            pl.BlockSpec((D, E), lambda i: (0, 0)),
            pl.BlockSpec((1, E), lambda i: (0, 0)),
        ],
        out_specs=[
            pl.BlockSpec((TB, K), lambda i: (i, 0)),
            pl.BlockSpec((TB, K), lambda i: (i, 0)),
            pl.BlockSpec((1, E), lambda i: (0, 0)),
            pl.BlockSpec((1, E), lambda i: (0, 0)),
            pl.BlockSpec((1, 1), lambda i: (0, 0)),
        ],
        out_shape=[
            jax.ShapeDtypeStruct((T, K), jnp.float32),
            jax.ShapeDtypeStruct((T, K), jnp.int32),
            jax.ShapeDtypeStruct((1, E), jnp.float32),
            jax.ShapeDtypeStruct((1, E), jnp.float32),
            jax.ShapeDtypeStruct((1, 1), jnp.float32),
        ],
        scratch_shapes=[
            pltpu.VMEM((NBUF, TB, D), jnp.float32),
            pltpu.SemaphoreType.DMA((NBUF,)),
        ],
    )(x_flat, W, b.reshape(1, E))
    return tkw, tki.astype(jnp.int64), aux[0, 0]


def kernel(x_flat, W, b):
    return _router(x_flat, W, b)
